# baseline (speedup 1.0000x reference)
import dataclasses
import functools

import jax
import jax.numpy as jnp
from jax import lax
from jax.experimental import pallas as pl
from jax.experimental.pallas import tpu as pltpu
from jax.experimental.pallas import tpu_sc as plsc

_LANES = 16


@functools.lru_cache(maxsize=None)
def _build_gather(batch: int, vocab: int):
    mesh = plsc.VectorSubcoreMesh(core_axis_name="c", subcore_axis_name="s",
                                  num_cores=1, num_subcores=8)
    nc, ns = mesh.num_cores, mesh.num_subcores
    nw = nc * ns
    assert batch % (8 * nw) == 0
    b_per_w = batch // nw

    cp = pltpu.CompilerParams()
    if "needs_layout_passes" in pltpu.CompilerParams.__dataclass_fields__:
        cp = dataclasses.replace(cp, needs_layout_passes=False)

    @functools.partial(
        pl.kernel,
        out_type=jax.ShapeDtypeStruct((batch,), jnp.float32),
        mesh=mesh,
        compiler_params=cp,
        scratch_types=[
            pltpu.VMEM((vocab,), jnp.float32),
            pltpu.VMEM((b_per_w,), jnp.int32),
            pltpu.VMEM((b_per_w,), jnp.float32),
            pltpu.SemaphoreType.DMA,
            pltpu.SemaphoreType.DMA,
        ],
    )
    def lookup(table_hbm, idx_hbm, out_hbm, tab_v, idx_v, out_v, sem_t, sem_i):
        wid = lax.axis_index("s") * nc + lax.axis_index("c")
        base = wid * b_per_w
        copy_t = pltpu.async_copy(table_hbm, tab_v, sem_t)
        copy_i = pltpu.async_copy(idx_hbm.at[pl.ds(base, b_per_w)], idx_v, sem_i)
        copy_i.wait()
        copy_t.wait()

        for j in range(b_per_w // _LANES):
            idx = idx_v[pl.ds(j * _LANES, _LANES)]
            out_v[pl.ds(j * _LANES, _LANES)] = plsc.load_gather(tab_v, [idx])

        pltpu.sync_copy(out_v, out_hbm.at[pl.ds(base, b_per_w)])

    return lookup


@jax.jit
def kernel(y, embed_weight):
    batch = y.shape[0]
    vocab = embed_weight.shape[0]
    table = embed_weight.reshape(vocab)
    out = _build_gather(batch, vocab)(table, y.astype(jnp.int32))
    return out.reshape(batch, 1)


# speedup vs baseline: 1.0380x; 1.0380x over previous
import dataclasses
import functools

import jax
import jax.numpy as jnp
from jax import lax
from jax.experimental import pallas as pl
from jax.experimental.pallas import tpu as pltpu
from jax.experimental.pallas import tpu_sc as plsc

_LANES = 16
_CHUNKS = 4


@functools.lru_cache(maxsize=None)
def _build_gather(batch: int, vocab: int):
    mesh = plsc.VectorSubcoreMesh(core_axis_name="c", subcore_axis_name="s",
                                  num_cores=1)
    nc, ns = mesh.num_cores, mesh.num_subcores
    nw = nc * ns
    assert batch % (8 * nw) == 0
    b_per_w = batch // nw
    chunk = b_per_w // _CHUNKS
    assert chunk % (8 * _LANES) == 0

    cp = pltpu.CompilerParams()
    if "needs_layout_passes" in pltpu.CompilerParams.__dataclass_fields__:
        cp = dataclasses.replace(cp, needs_layout_passes=False)

    @functools.partial(
        pl.kernel,
        out_type=jax.ShapeDtypeStruct((batch,), jnp.float32),
        mesh=mesh,
        compiler_params=cp,
        scratch_types=[
            pltpu.VMEM((vocab,), jnp.float32),
            pltpu.VMEM((b_per_w,), jnp.int32),
            pltpu.VMEM((b_per_w,), jnp.float32),
            pltpu.SemaphoreType.DMA,
            pltpu.SemaphoreType.DMA,
            pltpu.SemaphoreType.DMA,
        ],
    )
    def lookup(table_hbm, idx_hbm, out_hbm, tab_v, idx_v, out_v,
               sem_t, sem_i, sem_o):
        wid = lax.axis_index("s") * nc + lax.axis_index("c")
        base = wid * b_per_w
        copy_t = pltpu.async_copy(table_hbm, tab_v, sem_t)
        copy_i = pltpu.async_copy(idx_hbm.at[pl.ds(base, b_per_w)], idx_v,
                                  sem_i)
        copy_i.wait()
        copy_t.wait()

        out_copies = []
        for c in range(_CHUNKS):
            lo = c * chunk
            for j in range(chunk // _LANES):
                off = lo + j * _LANES
                idx = idx_v[pl.ds(off, _LANES)]
                out_v[pl.ds(off, _LANES)] = plsc.load_gather(tab_v, [idx])
            out_copies.append(
                pltpu.async_copy(out_v.at[pl.ds(lo, chunk)],
                                 out_hbm.at[pl.ds(base + lo, chunk)], sem_o))
        for copy_o in out_copies:
            copy_o.wait()

    return lookup


@jax.jit
def kernel(y, embed_weight):
    batch = y.shape[0]
    vocab = embed_weight.shape[0]
    table = embed_weight.reshape(vocab)
    out = _build_gather(batch, vocab)(table, y.astype(jnp.int32))
    return out.reshape(batch, 1)
